# baseline (speedup 1.0000x reference)
import functools

import jax
import jax.numpy as jnp
from jax import lax
from jax.experimental import pallas as pl
from jax.experimental.pallas import tpu as pltpu
from jax.experimental.pallas import tpu_sc as plsc

N = 10000
D = 128
NPAD = 10112
NTILES = 32
CHUNK = 80
NBUF = 4
LEAK = 0.2
ROWB = 400
GRID = N // ROWB
DBLK = 128


def _leaky(z):
    return jnp.maximum(z, LEAK * z)


_GATHER_DNUMS = lax.GatherDimensionNumbers(
    offset_dims=(), collapsed_slice_dims=(0,), start_index_map=(0,))


def _bcast_lane(v16, i):
    idx = jnp.full((16, 1), i, jnp.int32)
    return lax.gather(v16, idx, _GATHER_DNUMS, (1,),
                      mode=lax.GatherScatterMode.PROMISE_IN_BOUNDS)


def _tc_transform_body(x_ref, w_ref, as_ref, ad_ref, h_ref, s_ref, d_ref):
    h = jnp.dot(x_ref[...], w_ref[...], preferred_element_type=jnp.float32)
    h_ref[...] = h
    s_ref[...] = jnp.dot(h, as_ref[...], preferred_element_type=jnp.float32)
    d_ref[...] = jnp.dot(h, ad_ref[...], preferred_element_type=jnp.float32)


def _tc_transform(x, W, a_s, a_d):
    return pl.pallas_call(
        _tc_transform_body,
        grid=(GRID,),
        in_specs=[
            pl.BlockSpec((ROWB, D), lambda i: (i, 0)),
            pl.BlockSpec((D, D), lambda i: (0, 0)),
            pl.BlockSpec((D, 1), lambda i: (0, 0)),
            pl.BlockSpec((D, 1), lambda i: (0, 0)),
        ],
        out_specs=[
            pl.BlockSpec((ROWB, D), lambda i: (i, 0)),
            pl.BlockSpec((ROWB, 1), lambda i: (i, 0)),
            pl.BlockSpec((ROWB, 1), lambda i: (i, 0)),
        ],
        out_shape=[
            jax.ShapeDtypeStruct((N, D), jnp.float32),
            jax.ShapeDtypeStruct((N, 1), jnp.float32),
            jax.ShapeDtypeStruct((N, 1), jnp.float32),
        ],
    )(x, W, a_s.reshape(D, 1), a_d.reshape(D, 1))


def _tc_combine_transform_body(acc_ref, dinv_ref, b_ref, w_ref, as_ref,
                               ad_ref, h_ref, s_ref, d_ref):
    agg = (acc_ref[0] + acc_ref[1]) * dinv_ref[...]
    x = jnp.maximum(agg + b_ref[...], 0.0)
    h = jnp.dot(x, w_ref[...], preferred_element_type=jnp.float32)
    h_ref[...] = h
    s_ref[...] = jnp.dot(h, as_ref[...], preferred_element_type=jnp.float32)
    d_ref[...] = jnp.dot(h, ad_ref[...], preferred_element_type=jnp.float32)


def _tc_combine_transform(acc, dinv, b, W, a_s, a_d):
    return pl.pallas_call(
        _tc_combine_transform_body,
        grid=(GRID,),
        in_specs=[
            pl.BlockSpec((2, ROWB, D), lambda i: (0, i, 0)),
            pl.BlockSpec((ROWB, 1), lambda i: (i, 0)),
            pl.BlockSpec((1, D), lambda i: (0, 0)),
            pl.BlockSpec((D, D), lambda i: (0, 0)),
            pl.BlockSpec((D, 1), lambda i: (0, 0)),
            pl.BlockSpec((D, 1), lambda i: (0, 0)),
        ],
        out_specs=[
            pl.BlockSpec((ROWB, D), lambda i: (i, 0)),
            pl.BlockSpec((ROWB, 1), lambda i: (i, 0)),
            pl.BlockSpec((ROWB, 1), lambda i: (i, 0)),
        ],
        out_shape=[
            jax.ShapeDtypeStruct((N, D), jnp.float32),
            jax.ShapeDtypeStruct((N, 1), jnp.float32),
            jax.ShapeDtypeStruct((N, 1), jnp.float32),
        ],
    )(acc, dinv, b.reshape(1, D), W, a_s.reshape(D, 1), a_d.reshape(D, 1))


def _tc_combine_body(acc_ref, dinv_ref, b_ref, o_ref):
    o_ref[...] = (acc_ref[0] + acc_ref[1]) * dinv_ref[...] + b_ref[...]


def _tc_combine(acc, dinv, b):
    return pl.pallas_call(
        _tc_combine_body,
        grid=(GRID,),
        in_specs=[
            pl.BlockSpec((2, ROWB, D), lambda i: (0, i, 0)),
            pl.BlockSpec((ROWB, 1), lambda i: (i, 0)),
            pl.BlockSpec((1, D), lambda i: (0, 0)),
        ],
        out_specs=pl.BlockSpec((ROWB, D), lambda i: (i, 0)),
        out_shape=jax.ShapeDtypeStruct((N, D), jnp.float32),
    )(acc, dinv, b.reshape(1, D))


def _tc_dinv_body(den_ref, o_ref):
    o_ref[...] = 1.0 / (jnp.sum(den_ref[...], axis=0, keepdims=True) + 1e-16)


def _tc_dinv(den_parts):
    out = pl.pallas_call(
        _tc_dinv_body,
        grid=(NPAD // DBLK,),
        in_specs=[pl.BlockSpec((NTILES, DBLK), lambda i: (0, i))],
        out_specs=pl.BlockSpec((1, DBLK), lambda i: (0, i)),
        out_shape=jax.ShapeDtypeStruct((1, NPAD), jnp.float32),
    )(den_parts)
    return out.reshape(NPAD, 1)


def _make_weight_kernel(per_tile, e_pad):
    mesh = plsc.VectorSubcoreMesh(core_axis_name="c", subcore_axis_name="s")

    @functools.partial(
        pl.kernel,
        mesh=mesh,
        compiler_params=pltpu.CompilerParams(needs_layout_passes=False),
        out_type=[
            jax.ShapeDtypeStruct((e_pad,), jnp.float32),
            jax.ShapeDtypeStruct((NTILES, NPAD), jnp.float32),
        ],
        scratch_types=[
            pltpu.VMEM((NPAD,), jnp.float32),
            pltpu.VMEM((NPAD,), jnp.float32),
            pltpu.VMEM((NPAD,), jnp.float32),
            pltpu.VMEM((per_tile,), jnp.int32),
            pltpu.VMEM((per_tile,), jnp.int32),
            pltpu.VMEM((per_tile,), jnp.float32),
            pltpu.VMEM((16,), jnp.float32),
        ],
    )
    def weight_kernel(asrc_hbm, adst_hbm, gv_hbm, src_hbm, dst_hbm,
                      w_out, den_out,
                      asrc_v, adst_v, den_v, src_v, dst_v, w_v, gv_v):
        cid = lax.axis_index("c")
        sid = lax.axis_index("s")
        tid = cid * 16 + sid

        pltpu.sync_copy(asrc_hbm, asrc_v)
        pltpu.sync_copy(adst_hbm, adst_v)
        pltpu.sync_copy(gv_hbm, gv_v)
        pltpu.sync_copy(src_hbm.at[pl.ds(tid * per_tile, per_tile)], src_v)
        pltpu.sync_copy(dst_hbm.at[pl.ds(tid * per_tile, per_tile)], dst_v)

        zero16 = jnp.zeros((16,), jnp.float32)

        def _zero(r, carry):
            den_v[pl.ds(r * 16, 16)] = zero16
            return carry

        lax.fori_loop(0, NPAD // 16, _zero, 0)

        gv = gv_v[...]

        def _group(g, carry):
            sv = src_v[pl.ds(g * 16, 16)]
            dv = dst_v[pl.ds(g * 16, 16)]
            a1 = plsc.load_gather(asrc_v, [sv])
            a2 = plsc.load_gather(adst_v, [dv])
            e = _leaky(a1 + a2)
            bound = _leaky(gv + a2)
            w = jnp.exp(e - bound)
            w_v[pl.ds(g * 16, 16)] = w
            plsc.addupdate_scatter(den_v, [dv], w)
            return carry

        lax.fori_loop(0, per_tile // 16, _group, 0)

        pltpu.sync_copy(w_v, w_out.at[pl.ds(tid * per_tile, per_tile)])
        pltpu.sync_copy(den_v, den_out.at[tid])

    return weight_kernel


def _make_agg_kernel(nchunks):
    mesh = plsc.VectorSubcoreMesh(core_axis_name="c", subcore_axis_name="s")
    rows_per_tile = NPAD // 16

    @functools.partial(
        pl.kernel,
        mesh=mesh,
        compiler_params=pltpu.CompilerParams(needs_layout_passes=False),
        out_type=jax.ShapeDtypeStruct((2, NPAD, D), jnp.float32),
        scratch_types=[
            pltpu.VMEM((NBUF, CHUNK), jnp.int32),
            pltpu.VMEM((NBUF, CHUNK), jnp.int32),
            pltpu.VMEM((NBUF, CHUNK), jnp.float32),
            pltpu.VMEM((NBUF, CHUNK, D), jnp.float32),
            pltpu.VMEM_SHARED((NPAD, D), jnp.float32),
        ] + [pltpu.SemaphoreType.DMA] * (3 * NBUF),
    )
    def agg_kernel(h_hbm, w_hbm, src_hbm, dst_hbm, acc_out,
                   src_v, dst_v, w_v, rows_v, acc_s, *sems):
        semi = sems[0:NBUF]
        semg = sems[NBUF:2 * NBUF]
        sems_ = sems[2 * NBUF:3 * NBUF]
        cid = lax.axis_index("c")
        sid = lax.axis_index("s")
        tid = cid * 16 + sid
        base = sid * rows_per_tile
        crow0 = tid * nchunks

        zero16 = jnp.zeros((16,), jnp.float32)

        def _zero_row(r, carry):
            for j in range(D // 16):
                rows_v[0, r, pl.ds(j * 16, 16)] = zero16
            return carry

        lax.fori_loop(0, CHUNK, _zero_row, 0)

        nfull = rows_per_tile // CHUNK
        for k in range(nfull):
            pltpu.sync_copy(rows_v.at[0],
                            acc_s.at[pl.ds(base + k * CHUNK, CHUNK)])
        tail = rows_per_tile % CHUNK
        if tail:
            pltpu.sync_copy(
                rows_v.at[0, pl.ds(0, tail)],
                acc_s.at[pl.ds(base + rows_per_tile - tail, tail)])
        plsc.subcore_barrier()

        def _issue_idx(t, p):
            pltpu.async_copy(src_hbm.at[crow0 + t], src_v.at[p], semi[p])
            pltpu.async_copy(dst_hbm.at[crow0 + t], dst_v.at[p], semi[p])
            pltpu.async_copy(w_hbm.at[crow0 + t], w_v.at[p], semi[p])

        def _wait_idx(t, p):
            pltpu.make_async_copy(src_hbm.at[crow0 + t], src_v.at[p],
                                  semi[p]).wait()
            pltpu.make_async_copy(dst_hbm.at[crow0 + t], dst_v.at[p],
                                  semi[p]).wait()
            pltpu.make_async_copy(w_hbm.at[crow0 + t], w_v.at[p],
                                  semi[p]).wait()

        def _issue_gather(p):
            pltpu.async_copy(h_hbm.at[src_v.at[p]], rows_v.at[p], semg[p])

        def _wait_gather(p):
            pltpu.make_async_copy(h_hbm.at[src_v.at[p]], rows_v.at[p],
                                  semg[p]).wait()

        def _issue_scatter(p):
            pltpu.async_copy(rows_v.at[p], acc_s.at[dst_v.at[p]], sems_[p],
                             add=True)

        def _wait_scatter(p):
            pltpu.make_async_copy(rows_v.at[p], acc_s.at[dst_v.at[p]],
                                  sems_[p]).wait()

        _issue_idx(0, 0)
        _wait_idx(0, 0)
        _issue_gather(0)
        _issue_idx(1, 1)

        def _body(b, carry):
            for p in range(NBUF):
                t = b * NBUF + p
                pn = (p + 1) % NBUF
                pnn = (p + 2) % NBUF

                @pl.when(t >= 2)
                def _():
                    _wait_scatter(pnn)

                @pl.when(t < nchunks - 2)
                def _():
                    _issue_idx(t + 2, pnn)

                @pl.when(t < nchunks - 1)
                def _():
                    _wait_idx(t + 1, pn)
                    _issue_gather(pn)

                _wait_gather(p)

                def _scale(g, carry2):
                    w16 = w_v[p, pl.ds(g * 16, 16)]
                    for i in range(16):
                        wi = _bcast_lane(w16, i)
                        r = g * 16 + i
                        for j in range(D // 16):
                            rows_v[p, r, pl.ds(j * 16, 16)] = (
                                rows_v[p, r, pl.ds(j * 16, 16)] * wi)
                    return carry2

                lax.fori_loop(0, CHUNK // 16, _scale, 0)
                _issue_scatter(p)

            return carry

        lax.fori_loop(0, nchunks // NBUF, _body, 0)
        _wait_scatter((nchunks - 2) % NBUF)
        _wait_scatter((nchunks - 1) % NBUF)
        plsc.subcore_barrier()

        pltpu.sync_copy(acc_s.at[pl.ds(base, rows_per_tile)],
                        acc_out.at[cid, pl.ds(base, rows_per_tile)])

    return agg_kernel


def _pad_alpha(a):
    return jnp.concatenate([a.reshape(-1), jnp.zeros((NPAD - N,), jnp.float32)])


def kernel(x, edge_index, W0, as0, ad0, b0, W1, as1, ad1, b1, W2, as2, ad2, b2):
    n = x.shape[0]
    e_real = edge_index.shape[1] + n
    nch = -(-e_real // (NTILES * CHUNK * NBUF)) * NBUF
    per_tile = nch * CHUNK
    e_pad = NTILES * per_tile

    loops = jnp.arange(n, dtype=edge_index.dtype)
    src = jnp.concatenate(
        [edge_index[0], loops, jnp.zeros((e_pad - e_real,), edge_index.dtype)])
    dst = jnp.concatenate(
        [edge_index[1], loops,
         jnp.full((e_pad - e_real,), n, edge_index.dtype)])
    src2d = src.reshape(NTILES * nch, CHUNK)
    dst2d = dst.reshape(NTILES * nch, CHUNK)

    weight_kernel = _make_weight_kernel(per_tile, e_pad)
    agg_kernel = _make_agg_kernel(nch)

    def layer_edges(h, s, d):
        gv = jnp.full((16,), jnp.max(s), jnp.float32)
        w, den_parts = weight_kernel(_pad_alpha(s), _pad_alpha(d), gv, src, dst)
        dinv = _tc_dinv(den_parts)[:n]
        acc = agg_kernel(h, w.reshape(NTILES * nch, CHUNK), src2d, dst2d)
        return acc, dinv

    h1, s1, d1 = _tc_transform(x, W0, as0, ad0)
    acc1, dinv1 = layer_edges(h1, s1, d1)
    h2, s2, d2 = _tc_combine_transform(acc1, dinv1, b0, W1, as1, ad1)
    acc2, dinv2 = layer_edges(h2, s2, d2)
    h3, s3, d3 = _tc_combine_transform(acc2, dinv2, b1, W2, as2, ad2)
    acc3, dinv3 = layer_edges(h3, s3, d3)
    return _tc_combine(acc3, dinv3, b2)


# speedup vs baseline: 1.7783x; 1.7783x over previous
import functools

import jax
import jax.numpy as jnp
from jax import lax
from jax.experimental import pallas as pl
from jax.experimental.pallas import tpu as pltpu
from jax.experimental.pallas import tpu_sc as plsc

N = 10000
D = 128
NPAD = 10112
NTILES = 32
CHUNK = 96
NBUF = 3
LEAK = 0.2
ROWB = 400
GRID = N // ROWB
DBLK = 128


def _leaky(z):
    return jnp.maximum(z, LEAK * z)


_GATHER_DNUMS = lax.GatherDimensionNumbers(
    offset_dims=(), collapsed_slice_dims=(0,), start_index_map=(0,))


def _bcast_lane(v16, i):
    idx = jnp.full((16, 1), i, jnp.int32)
    return lax.gather(v16, idx, _GATHER_DNUMS, (1,),
                      mode=lax.GatherScatterMode.PROMISE_IN_BOUNDS)


def _tc_transform_body(x_ref, w_ref, as_ref, ad_ref, h_ref, s_ref, d_ref):
    h = jnp.dot(x_ref[...], w_ref[...], preferred_element_type=jnp.float32)
    h_ref[...] = h
    s_ref[...] = jnp.dot(h, as_ref[...], preferred_element_type=jnp.float32)
    d_ref[...] = jnp.dot(h, ad_ref[...], preferred_element_type=jnp.float32)


def _tc_transform(x, W, a_s, a_d):
    return pl.pallas_call(
        _tc_transform_body,
        grid=(GRID,),
        in_specs=[
            pl.BlockSpec((ROWB, D), lambda i: (i, 0)),
            pl.BlockSpec((D, D), lambda i: (0, 0)),
            pl.BlockSpec((D, 1), lambda i: (0, 0)),
            pl.BlockSpec((D, 1), lambda i: (0, 0)),
        ],
        out_specs=[
            pl.BlockSpec((ROWB, D), lambda i: (i, 0)),
            pl.BlockSpec((ROWB, 1), lambda i: (i, 0)),
            pl.BlockSpec((ROWB, 1), lambda i: (i, 0)),
        ],
        out_shape=[
            jax.ShapeDtypeStruct((N, D), jnp.float32),
            jax.ShapeDtypeStruct((N, 1), jnp.float32),
            jax.ShapeDtypeStruct((N, 1), jnp.float32),
        ],
    )(x, W, a_s.reshape(D, 1), a_d.reshape(D, 1))


def _tc_combine_transform_body(acc_ref, dinv_ref, b_ref, w_ref, as_ref,
                               ad_ref, h_ref, s_ref, d_ref):
    agg = (acc_ref[0] + acc_ref[1]) * dinv_ref[...]
    x = jnp.maximum(agg + b_ref[...], 0.0)
    h = jnp.dot(x, w_ref[...], preferred_element_type=jnp.float32)
    h_ref[...] = h
    s_ref[...] = jnp.dot(h, as_ref[...], preferred_element_type=jnp.float32)
    d_ref[...] = jnp.dot(h, ad_ref[...], preferred_element_type=jnp.float32)


def _tc_combine_transform(acc, dinv, b, W, a_s, a_d):
    return pl.pallas_call(
        _tc_combine_transform_body,
        grid=(GRID,),
        in_specs=[
            pl.BlockSpec((2, ROWB, D), lambda i: (0, i, 0)),
            pl.BlockSpec((ROWB, 1), lambda i: (i, 0)),
            pl.BlockSpec((1, D), lambda i: (0, 0)),
            pl.BlockSpec((D, D), lambda i: (0, 0)),
            pl.BlockSpec((D, 1), lambda i: (0, 0)),
            pl.BlockSpec((D, 1), lambda i: (0, 0)),
        ],
        out_specs=[
            pl.BlockSpec((ROWB, D), lambda i: (i, 0)),
            pl.BlockSpec((ROWB, 1), lambda i: (i, 0)),
            pl.BlockSpec((ROWB, 1), lambda i: (i, 0)),
        ],
        out_shape=[
            jax.ShapeDtypeStruct((N, D), jnp.float32),
            jax.ShapeDtypeStruct((N, 1), jnp.float32),
            jax.ShapeDtypeStruct((N, 1), jnp.float32),
        ],
    )(acc, dinv, b.reshape(1, D), W, a_s.reshape(D, 1), a_d.reshape(D, 1))


def _tc_combine_body(acc_ref, dinv_ref, b_ref, o_ref):
    o_ref[...] = (acc_ref[0] + acc_ref[1]) * dinv_ref[...] + b_ref[...]


def _tc_combine(acc, dinv, b):
    return pl.pallas_call(
        _tc_combine_body,
        grid=(GRID,),
        in_specs=[
            pl.BlockSpec((2, ROWB, D), lambda i: (0, i, 0)),
            pl.BlockSpec((ROWB, 1), lambda i: (i, 0)),
            pl.BlockSpec((1, D), lambda i: (0, 0)),
        ],
        out_specs=pl.BlockSpec((ROWB, D), lambda i: (i, 0)),
        out_shape=jax.ShapeDtypeStruct((N, D), jnp.float32),
    )(acc, dinv, b.reshape(1, D))


def _tc_dinv_body(den_ref, o_ref):
    o_ref[...] = 1.0 / (jnp.sum(den_ref[...], axis=0, keepdims=True) + 1e-16)


def _tc_dinv(den_parts):
    out = pl.pallas_call(
        _tc_dinv_body,
        grid=(NPAD // DBLK,),
        in_specs=[pl.BlockSpec((NTILES, DBLK), lambda i: (0, i))],
        out_specs=pl.BlockSpec((1, DBLK), lambda i: (0, i)),
        out_shape=jax.ShapeDtypeStruct((1, NPAD), jnp.float32),
    )(den_parts)
    return out.reshape(NPAD, 1)


def _make_weight_kernel(per_tile, e_pad):
    mesh = plsc.VectorSubcoreMesh(core_axis_name="c", subcore_axis_name="s")

    @functools.partial(
        pl.kernel,
        mesh=mesh,
        compiler_params=pltpu.CompilerParams(needs_layout_passes=False),
        out_type=[
            jax.ShapeDtypeStruct((e_pad,), jnp.float32),
            jax.ShapeDtypeStruct((NTILES, NPAD), jnp.float32),
        ],
        scratch_types=[
            pltpu.VMEM((NPAD,), jnp.float32),
            pltpu.VMEM((NPAD,), jnp.float32),
            pltpu.VMEM((NPAD,), jnp.float32),
            pltpu.VMEM((per_tile,), jnp.int32),
            pltpu.VMEM((per_tile,), jnp.int32),
            pltpu.VMEM((per_tile,), jnp.float32),
            pltpu.VMEM((16,), jnp.float32),
        ],
    )
    def weight_kernel(asrc_hbm, adst_hbm, gv_hbm, src_hbm, dst_hbm,
                      w_out, den_out,
                      asrc_v, adst_v, den_v, src_v, dst_v, w_v, gv_v):
        cid = lax.axis_index("c")
        sid = lax.axis_index("s")
        tid = cid * 16 + sid

        pltpu.sync_copy(asrc_hbm, asrc_v)
        pltpu.sync_copy(adst_hbm, adst_v)
        pltpu.sync_copy(gv_hbm, gv_v)
        pltpu.sync_copy(src_hbm.at[pl.ds(tid * per_tile, per_tile)], src_v)
        pltpu.sync_copy(dst_hbm.at[pl.ds(tid * per_tile, per_tile)], dst_v)

        zero16 = jnp.zeros((16,), jnp.float32)

        def _zero(r, carry):
            den_v[pl.ds(r * 16, 16)] = zero16
            return carry

        lax.fori_loop(0, NPAD // 16, _zero, 0)

        gv = gv_v[...]

        def _group(g, carry):
            sv = src_v[pl.ds(g * 16, 16)]
            dv = dst_v[pl.ds(g * 16, 16)]
            a1 = plsc.load_gather(asrc_v, [sv])
            a2 = plsc.load_gather(adst_v, [dv])
            e = _leaky(a1 + a2)
            bound = _leaky(gv + a2)
            w = jnp.exp(e - bound)
            w_v[pl.ds(g * 16, 16)] = w
            plsc.addupdate_scatter(den_v, [dv], w)
            return carry

        lax.fori_loop(0, per_tile // 16, _group, 0)

        pltpu.sync_copy(w_v, w_out.at[pl.ds(tid * per_tile, per_tile)])
        pltpu.sync_copy(den_v, den_out.at[tid])

    return weight_kernel


def _make_agg_kernel(nch0, nch1):
    mesh = plsc.VectorSubcoreMesh(core_axis_name="c", subcore_axis_name="s")
    rows_per_tile = NPAD // 16
    assert nch0 % NBUF == 0 and nch1 % NBUF == 0

    @functools.partial(
        pl.kernel,
        mesh=mesh,
        compiler_params=pltpu.CompilerParams(needs_layout_passes=False),
        out_type=jax.ShapeDtypeStruct((2, NPAD, D), jnp.float32),
        scratch_types=[
            pltpu.VMEM((NBUF, CHUNK), jnp.int32),
            pltpu.VMEM((NBUF, CHUNK), jnp.int32),
            pltpu.VMEM((NBUF, CHUNK), jnp.float32),
            pltpu.VMEM((NBUF, CHUNK, D), jnp.float32),
            pltpu.VMEM_SHARED((NPAD, D), jnp.float32),
        ] + [pltpu.SemaphoreType.DMA] * (3 * NBUF),
    )
    def agg_kernel(h_hbm, w_hbm, src_hbm, dst_hbm, acc_out,
                   src_v, dst_v, w_v, rows_v, acc_s, *sems):
        semi = sems[0:NBUF]
        semg = sems[NBUF:2 * NBUF]
        sems_ = sems[2 * NBUF:3 * NBUF]
        cid = lax.axis_index("c")
        sid = lax.axis_index("s")
        base = sid * rows_per_tile
        n_my = jnp.where(cid == 0, nch0, nch1)
        crow0 = jnp.where(cid == 0, sid * nch0, 16 * nch0 + sid * nch1)

        zero16 = jnp.zeros((16,), jnp.float32)

        def _zero_row(r, carry):
            for j in range(D // 16):
                rows_v[0, r, pl.ds(j * 16, 16)] = zero16
            return carry

        lax.fori_loop(0, CHUNK, _zero_row, 0)

        nfull = rows_per_tile // CHUNK
        for k in range(nfull):
            pltpu.sync_copy(rows_v.at[0],
                            acc_s.at[pl.ds(base + k * CHUNK, CHUNK)])
        tail = rows_per_tile % CHUNK
        if tail:
            pltpu.sync_copy(
                rows_v.at[0, pl.ds(0, tail)],
                acc_s.at[pl.ds(base + rows_per_tile - tail, tail)])
        plsc.subcore_barrier()

        def _issue_idx(t, p):
            pltpu.async_copy(src_hbm.at[crow0 + t], src_v.at[p], semi[p])
            pltpu.async_copy(dst_hbm.at[crow0 + t], dst_v.at[p], semi[p])
            pltpu.async_copy(w_hbm.at[crow0 + t], w_v.at[p], semi[p])

        def _wait_idx(t, p):
            pltpu.make_async_copy(src_hbm.at[crow0 + t], src_v.at[p],
                                  semi[p]).wait()
            pltpu.make_async_copy(dst_hbm.at[crow0 + t], dst_v.at[p],
                                  semi[p]).wait()
            pltpu.make_async_copy(w_hbm.at[crow0 + t], w_v.at[p],
                                  semi[p]).wait()

        def _issue_gather(p):
            pltpu.async_copy(h_hbm.at[src_v.at[p]], rows_v.at[p], semg[p])

        def _wait_gather(p):
            pltpu.make_async_copy(h_hbm.at[src_v.at[p]], rows_v.at[p],
                                  semg[p]).wait()

        def _issue_scatter(p):
            pltpu.async_copy(rows_v.at[p], acc_s.at[dst_v.at[p]], sems_[p],
                             add=True)

        def _wait_scatter(p):
            pltpu.make_async_copy(rows_v.at[p], acc_s.at[dst_v.at[p]],
                                  sems_[p]).wait()

        _issue_idx(0, 0)
        _wait_idx(0, 0)
        _issue_gather(0)
        _issue_idx(1, 1)

        def _body(b, carry):
            for p in range(NBUF):
                t = b * NBUF + p
                pn = (p + 1) % NBUF
                pnn = (p + 2) % NBUF

                @pl.when(t < n_my - 1)
                def _():
                    _wait_idx(t + 1, pn)
                    _issue_gather(pn)

                _wait_gather(p)

                def _scale(g, carry2):
                    w16 = w_v[p, pl.ds(g * 16, 16)]
                    for i in range(16):
                        wi = _bcast_lane(w16, i)
                        r = g * 16 + i
                        for j in range(D // 16):
                            rows_v[p, r, pl.ds(j * 16, 16)] = (
                                rows_v[p, r, pl.ds(j * 16, 16)] * wi)
                    return carry2

                lax.fori_loop(0, CHUNK // 16, _scale, 0)
                _issue_scatter(p)

                @pl.when(t >= 1)
                def _():
                    _wait_scatter(pnn)

                @pl.when(t < n_my - 2)
                def _():
                    _issue_idx(t + 2, pnn)

            return carry

        lax.fori_loop(0, n_my // NBUF, _body, 0)
        _wait_scatter(2)
        plsc.subcore_barrier()

        pltpu.sync_copy(acc_s.at[pl.ds(base, rows_per_tile)],
                        acc_out.at[cid, pl.ds(base, rows_per_tile)])

    return agg_kernel


def _pad_alpha(a):
    return jnp.concatenate([a.reshape(-1), jnp.zeros((NPAD - N,), jnp.float32)])


def kernel(x, edge_index, W0, as0, ad0, b0, W1, as1, ad1, b1, W2, as2, ad2, b2):
    n = x.shape[0]
    e_real = edge_index.shape[1] + n
    ctotal = -(-e_real // (16 * CHUNK * 2 * NBUF)) * 2 * NBUF
    nch0 = int(round(ctotal * 0.405 / NBUF)) * NBUF
    nch1 = ctotal - nch0
    e_pad = 16 * ctotal * CHUNK
    per_tile = e_pad // NTILES

    loops = jnp.arange(n, dtype=edge_index.dtype)
    src = jnp.concatenate(
        [edge_index[0], loops, jnp.zeros((e_pad - e_real,), edge_index.dtype)])
    dst = jnp.concatenate(
        [edge_index[1], loops,
         jnp.full((e_pad - e_real,), n, edge_index.dtype)])
    src2d = src.reshape(16 * ctotal, CHUNK)
    dst2d = dst.reshape(16 * ctotal, CHUNK)

    weight_kernel = _make_weight_kernel(per_tile, e_pad)
    agg_kernel = _make_agg_kernel(nch0, nch1)

    def layer_edges(h, s, d):
        gv = jnp.full((16,), jnp.max(s), jnp.float32)
        w, den_parts = weight_kernel(_pad_alpha(s), _pad_alpha(d), gv, src, dst)
        dinv = _tc_dinv(den_parts)[:n]
        acc = agg_kernel(h, w.reshape(16 * ctotal, CHUNK), src2d, dst2d)
        return acc, dinv

    h1, s1, d1 = _tc_transform(x, W0, as0, ad0)
    acc1, dinv1 = layer_edges(h1, s1, d1)
    h2, s2, d2 = _tc_combine_transform(acc1, dinv1, b0, W1, as1, ad1)
    acc2, dinv2 = layer_edges(h2, s2, d2)
    h3, s3, d3 = _tc_combine_transform(acc2, dinv2, b1, W2, as2, ad2)
    acc3, dinv3 = layer_edges(h3, s3, d3)
    return _tc_combine(acc3, dinv3, b2)


# speedup vs baseline: 2.0189x; 1.1353x over previous
import functools

import jax
import jax.numpy as jnp
from jax import lax
from jax.experimental import pallas as pl
from jax.experimental.pallas import tpu as pltpu
from jax.experimental.pallas import tpu_sc as plsc

N = 10000
D = 128
NPAD = 10112
NTILES = 32
CHUNK = 96
NBUF = 3
LEAK = 0.2
ROWB = 400
GRID = N // ROWB
DBLK = 128


def _leaky(z):
    return jnp.maximum(z, LEAK * z)


_GATHER_DNUMS = lax.GatherDimensionNumbers(
    offset_dims=(), collapsed_slice_dims=(0,), start_index_map=(0,))


def _bcast_lane(v16, i):
    idx = jnp.full((16, 1), i, jnp.int32)
    return lax.gather(v16, idx, _GATHER_DNUMS, (1,),
                      mode=lax.GatherScatterMode.PROMISE_IN_BOUNDS)


def _tc_transform_body(x_ref, w_ref, as_ref, ad_ref, h_ref, s_ref, d_ref):
    h = jnp.dot(x_ref[...], w_ref[...], preferred_element_type=jnp.float32)
    h_ref[...] = h
    s_ref[...] = jnp.dot(h, as_ref[...], preferred_element_type=jnp.float32)
    d_ref[...] = jnp.dot(h, ad_ref[...], preferred_element_type=jnp.float32)


def _tc_transform(x, W, a_s, a_d):
    return pl.pallas_call(
        _tc_transform_body,
        grid=(GRID,),
        in_specs=[
            pl.BlockSpec((ROWB, D), lambda i: (i, 0)),
            pl.BlockSpec((D, D), lambda i: (0, 0)),
            pl.BlockSpec((D, 1), lambda i: (0, 0)),
            pl.BlockSpec((D, 1), lambda i: (0, 0)),
        ],
        out_specs=[
            pl.BlockSpec((ROWB, D), lambda i: (i, 0)),
            pl.BlockSpec((ROWB, 1), lambda i: (i, 0)),
            pl.BlockSpec((ROWB, 1), lambda i: (i, 0)),
        ],
        out_shape=[
            jax.ShapeDtypeStruct((N, D), jnp.float32),
            jax.ShapeDtypeStruct((N, 1), jnp.float32),
            jax.ShapeDtypeStruct((N, 1), jnp.float32),
        ],
    )(x, W, a_s.reshape(D, 1), a_d.reshape(D, 1))


def _tc_combine_transform_body(acc_ref, dinv_ref, b_ref, w_ref, as_ref,
                               ad_ref, h_ref, s_ref, d_ref):
    agg = (acc_ref[0] + acc_ref[1]) * dinv_ref[...]
    x = jnp.maximum(agg + b_ref[...], 0.0)
    h = jnp.dot(x, w_ref[...], preferred_element_type=jnp.float32)
    h_ref[...] = h
    s_ref[...] = jnp.dot(h, as_ref[...], preferred_element_type=jnp.float32)
    d_ref[...] = jnp.dot(h, ad_ref[...], preferred_element_type=jnp.float32)


def _tc_combine_transform(acc, dinv, b, W, a_s, a_d):
    return pl.pallas_call(
        _tc_combine_transform_body,
        grid=(GRID,),
        in_specs=[
            pl.BlockSpec((2, ROWB, D), lambda i: (0, i, 0)),
            pl.BlockSpec((ROWB, 1), lambda i: (i, 0)),
            pl.BlockSpec((1, D), lambda i: (0, 0)),
            pl.BlockSpec((D, D), lambda i: (0, 0)),
            pl.BlockSpec((D, 1), lambda i: (0, 0)),
            pl.BlockSpec((D, 1), lambda i: (0, 0)),
        ],
        out_specs=[
            pl.BlockSpec((ROWB, D), lambda i: (i, 0)),
            pl.BlockSpec((ROWB, 1), lambda i: (i, 0)),
            pl.BlockSpec((ROWB, 1), lambda i: (i, 0)),
        ],
        out_shape=[
            jax.ShapeDtypeStruct((N, D), jnp.float32),
            jax.ShapeDtypeStruct((N, 1), jnp.float32),
            jax.ShapeDtypeStruct((N, 1), jnp.float32),
        ],
    )(acc, dinv, b.reshape(1, D), W, a_s.reshape(D, 1), a_d.reshape(D, 1))


def _tc_combine_body(acc_ref, dinv_ref, b_ref, o_ref):
    o_ref[...] = (acc_ref[0] + acc_ref[1]) * dinv_ref[...] + b_ref[...]


def _tc_combine(acc, dinv, b):
    return pl.pallas_call(
        _tc_combine_body,
        grid=(GRID,),
        in_specs=[
            pl.BlockSpec((2, ROWB, D), lambda i: (0, i, 0)),
            pl.BlockSpec((ROWB, 1), lambda i: (i, 0)),
            pl.BlockSpec((1, D), lambda i: (0, 0)),
        ],
        out_specs=pl.BlockSpec((ROWB, D), lambda i: (i, 0)),
        out_shape=jax.ShapeDtypeStruct((N, D), jnp.float32),
    )(acc, dinv, b.reshape(1, D))


def _tc_dinv_body(den_ref, o_ref):
    o_ref[...] = 1.0 / (jnp.sum(den_ref[...], axis=0, keepdims=True) + 1e-16)


def _tc_dinv(den_parts):
    out = pl.pallas_call(
        _tc_dinv_body,
        grid=(NPAD // DBLK,),
        in_specs=[pl.BlockSpec((NTILES, DBLK), lambda i: (0, i))],
        out_specs=pl.BlockSpec((1, DBLK), lambda i: (0, i)),
        out_shape=jax.ShapeDtypeStruct((1, NPAD), jnp.float32),
    )(den_parts)
    return out.reshape(NPAD, 1)


def _make_weight_kernel(per_tile, e_pad):
    mesh = plsc.VectorSubcoreMesh(core_axis_name="c", subcore_axis_name="s")

    @functools.partial(
        pl.kernel,
        mesh=mesh,
        compiler_params=pltpu.CompilerParams(needs_layout_passes=False),
        out_type=[
            jax.ShapeDtypeStruct((e_pad,), jnp.float32),
            jax.ShapeDtypeStruct((NTILES, NPAD), jnp.float32),
        ],
        scratch_types=[
            pltpu.VMEM((NPAD,), jnp.float32),
            pltpu.VMEM((NPAD,), jnp.float32),
            pltpu.VMEM((NPAD,), jnp.float32),
            pltpu.VMEM((per_tile,), jnp.int32),
            pltpu.VMEM((per_tile,), jnp.int32),
            pltpu.VMEM((per_tile,), jnp.float32),
            pltpu.VMEM((16,), jnp.float32),
        ],
    )
    def weight_kernel(asrc_hbm, adst_hbm, gv_hbm, src_hbm, dst_hbm,
                      w_out, den_out,
                      asrc_v, adst_v, den_v, src_v, dst_v, w_v, gv_v):
        cid = lax.axis_index("c")
        sid = lax.axis_index("s")
        tid = cid * 16 + sid

        pltpu.sync_copy(asrc_hbm, asrc_v)
        pltpu.sync_copy(adst_hbm, adst_v)
        pltpu.sync_copy(gv_hbm, gv_v)
        pltpu.sync_copy(src_hbm.at[pl.ds(tid * per_tile, per_tile)], src_v)
        pltpu.sync_copy(dst_hbm.at[pl.ds(tid * per_tile, per_tile)], dst_v)

        zero16 = jnp.zeros((16,), jnp.float32)

        def _zero(r, carry):
            den_v[pl.ds(r * 16, 16)] = zero16
            return carry

        lax.fori_loop(0, NPAD // 16, _zero, 0)

        gv = gv_v[...]

        def _group(g, carry):
            sv = src_v[pl.ds(g * 16, 16)]
            dv = dst_v[pl.ds(g * 16, 16)]
            a1 = plsc.load_gather(asrc_v, [sv])
            a2 = plsc.load_gather(adst_v, [dv])
            e = _leaky(a1 + a2)
            bound = _leaky(gv + a2)
            w = jnp.exp(e - bound)
            w_v[pl.ds(g * 16, 16)] = w
            plsc.addupdate_scatter(den_v, [dv], w)
            return carry

        lax.fori_loop(0, per_tile // 16, _group, 0)

        pltpu.sync_copy(w_v, w_out.at[pl.ds(tid * per_tile, per_tile)])
        pltpu.sync_copy(den_v, den_out.at[tid])

    return weight_kernel


def _make_agg_kernel(nch0, nch1):
    mesh = plsc.VectorSubcoreMesh(core_axis_name="c", subcore_axis_name="s")
    rows_per_tile = NPAD // 16
    assert nch0 % NBUF == 0 and nch1 % NBUF == 0

    @functools.partial(
        pl.kernel,
        mesh=mesh,
        compiler_params=pltpu.CompilerParams(needs_layout_passes=False),
        out_type=jax.ShapeDtypeStruct((2, NPAD, D), jnp.float32),
        scratch_types=[
            pltpu.VMEM((NBUF, CHUNK), jnp.int32),
            pltpu.VMEM((NBUF, CHUNK), jnp.int32),
            pltpu.VMEM((NBUF, CHUNK), jnp.float32),
            pltpu.VMEM((NBUF, CHUNK, D), jnp.float32),
            pltpu.VMEM_SHARED((NPAD, D), jnp.float32),
        ] + [pltpu.SemaphoreType.DMA] * (3 * NBUF),
    )
    def agg_kernel(h_hbm, w_hbm, src_hbm, dst_hbm, acc_out,
                   src_v, dst_v, w_v, rows_v, acc_s, *sems):
        semi = sems[0:NBUF]
        semg = sems[NBUF:2 * NBUF]
        sems_ = sems[2 * NBUF:3 * NBUF]
        cid = lax.axis_index("c")
        sid = lax.axis_index("s")
        base = sid * rows_per_tile
        n_my = jnp.where(cid == 0, nch0, nch1)
        crow0 = jnp.where(cid == 0, sid * nch0, 16 * nch0 + sid * nch1)

        zero16 = jnp.zeros((16,), jnp.float32)

        def _zero_row(r, carry):
            for j in range(D // 16):
                rows_v[0, r, pl.ds(j * 16, 16)] = zero16
            return carry

        lax.fori_loop(0, CHUNK, _zero_row, 0)

        nfull = rows_per_tile // CHUNK
        for k in range(nfull):
            pltpu.sync_copy(rows_v.at[0],
                            acc_s.at[pl.ds(base + k * CHUNK, CHUNK)])
        tail = rows_per_tile % CHUNK
        if tail:
            pltpu.sync_copy(
                rows_v.at[0, pl.ds(0, tail)],
                acc_s.at[pl.ds(base + rows_per_tile - tail, tail)])
        plsc.subcore_barrier()

        def _issue_idx(t, p):
            pltpu.async_copy(src_hbm.at[crow0 + t], src_v.at[p], semi[p])
            pltpu.async_copy(dst_hbm.at[crow0 + t], dst_v.at[p], semi[p])
            pltpu.async_copy(w_hbm.at[crow0 + t], w_v.at[p], semi[p])

        def _wait_idx(t, p):
            pltpu.make_async_copy(src_hbm.at[crow0 + t], src_v.at[p],
                                  semi[p]).wait()
            pltpu.make_async_copy(dst_hbm.at[crow0 + t], dst_v.at[p],
                                  semi[p]).wait()
            pltpu.make_async_copy(w_hbm.at[crow0 + t], w_v.at[p],
                                  semi[p]).wait()

        def _issue_gather(p):
            pltpu.async_copy(h_hbm.at[src_v.at[p]], rows_v.at[p], semg[p])

        def _wait_gather(p):
            pltpu.make_async_copy(h_hbm.at[src_v.at[p]], rows_v.at[p],
                                  semg[p]).wait()

        def _issue_scatter(p):
            pltpu.async_copy(rows_v.at[p], acc_s.at[dst_v.at[p]], sems_[p],
                             add=True)

        def _wait_scatter(p):
            pltpu.make_async_copy(rows_v.at[p], acc_s.at[dst_v.at[p]],
                                  sems_[p]).wait()

        _issue_idx(0, 0)
        _wait_idx(0, 0)
        _issue_gather(0)
        _issue_idx(1, 1)

        def _body(b, carry):
            for p in range(NBUF):
                t = b * NBUF + p
                pn = (p + 1) % NBUF
                pnn = (p + 2) % NBUF

                @pl.when(t < n_my - 1)
                def _():
                    _wait_idx(t + 1, pn)
                    _issue_gather(pn)

                _wait_gather(p)

                def _scale(g, carry2):
                    w16 = w_v[p, pl.ds(g * 16, 16)]
                    for i in range(16):
                        wi = _bcast_lane(w16, i)
                        r = g * 16 + i
                        for j in range(D // 16):
                            rows_v[p, r, pl.ds(j * 16, 16)] = (
                                rows_v[p, r, pl.ds(j * 16, 16)] * wi)
                    return carry2

                lax.fori_loop(0, CHUNK // 16, _scale, 0)
                _issue_scatter(p)

                @pl.when(t >= 1)
                def _():
                    _wait_scatter(pnn)

                @pl.when(t < n_my - 2)
                def _():
                    _issue_idx(t + 2, pnn)

            return carry

        lax.fori_loop(0, n_my // NBUF, _body, 0)
        _wait_scatter(2)
        plsc.subcore_barrier()

        pltpu.sync_copy(acc_s.at[pl.ds(base, rows_per_tile)],
                        acc_out.at[cid, pl.ds(base, rows_per_tile)])

    return agg_kernel


def _pad_alpha(a):
    return jnp.concatenate([a.reshape(-1), jnp.zeros((NPAD - N,), jnp.float32)])


def kernel(x, edge_index, W0, as0, ad0, b0, W1, as1, ad1, b1, W2, as2, ad2, b2):
    n = x.shape[0]
    e_real = edge_index.shape[1] + n
    ctotal = -(-e_real // (16 * CHUNK * 2 * NBUF)) * 2 * NBUF
    nch0 = int(round(ctotal * 0.595 / NBUF)) * NBUF
    nch1 = ctotal - nch0
    e_pad = 16 * ctotal * CHUNK
    per_tile = e_pad // NTILES

    loops = jnp.arange(n, dtype=edge_index.dtype)
    src = jnp.concatenate(
        [edge_index[0], loops, jnp.zeros((e_pad - e_real,), edge_index.dtype)])
    dst = jnp.concatenate(
        [edge_index[1], loops,
         jnp.full((e_pad - e_real,), n, edge_index.dtype)])
    src2d = src.reshape(16 * ctotal, CHUNK)
    dst2d = dst.reshape(16 * ctotal, CHUNK)

    weight_kernel = _make_weight_kernel(per_tile, e_pad)
    agg_kernel = _make_agg_kernel(nch0, nch1)

    def layer_edges(h, s, d):
        gv = jnp.full((16,), jnp.max(s), jnp.float32)
        w, den_parts = weight_kernel(_pad_alpha(s), _pad_alpha(d), gv, src, dst)
        dinv = _tc_dinv(den_parts)[:n]
        acc = agg_kernel(h, w.reshape(16 * ctotal, CHUNK), src2d, dst2d)
        return acc, dinv

    h1, s1, d1 = _tc_transform(x, W0, as0, ad0)
    acc1, dinv1 = layer_edges(h1, s1, d1)
    h2, s2, d2 = _tc_combine_transform(acc1, dinv1, b0, W1, as1, ad1)
    acc2, dinv2 = layer_edges(h2, s2, d2)
    h3, s3, d3 = _tc_combine_transform(acc2, dinv2, b1, W2, as2, ad2)
    acc3, dinv3 = layer_edges(h3, s3, d3)
    return _tc_combine(acc3, dinv3, b2)
